# baseline (speedup 1.0000x reference)
import functools

import jax
import jax.numpy as jnp
from jax.experimental import pallas as pl
from jax.experimental.pallas import tpu as pltpu


def _add_kernel(in_ref, tab_ref, out_ref):
    out_ref[...] = in_ref[...] + tab_ref[...]


@functools.partial(jax.jit, static_argnames=("block",))
def _posemb_add(inputs, table, block=2048):
    batch, seq, dim = inputs.shape
    grid = (seq // block, batch)
    return pl.pallas_call(
        _add_kernel,
        grid=grid,
        in_specs=[
            pl.BlockSpec((1, block, dim), lambda s, b: (b, s, 0)),
            pl.BlockSpec((block, dim), lambda s, b: (s, 0)),
        ],
        out_specs=pl.BlockSpec((1, block, dim), lambda s, b: (b, s, 0)),
        out_shape=jax.ShapeDtypeStruct(inputs.shape, inputs.dtype),
        compiler_params=pltpu.CompilerParams(
            vmem_limit_bytes=128 * 1024 * 1024,
        ),
    )(inputs, table)


def kernel(inputs, table):
    return _posemb_add(inputs, table)
